# baseline (speedup 1.0000x reference)
import functools

import numpy as np
import jax
import jax.numpy as jnp
from jax import lax
from jax.experimental import pallas as pl
from jax.experimental.pallas import tpu as pltpu
from jax.experimental.pallas import tpu_sc as plsc

IMG = 224
PIX = IMG * IMG
NB = 64
NCH = 3
NWORK = 32
TASKS = NB * NCH
TPW = TASKS // NWORK
ROWS_PER_CHUNK = 8
P = IMG * ROWS_PER_CHUNK
NCHUNK = PIX // P
LO = np.float32(0.001)
HI = np.float32(IMG - 1.001)
D0BIT = np.int32((224 << 24) - (1 << 32))
D1BIT = np.int32(1 << 16)


def _pack_bf16_pair(lo, hi):
    ul = lax.bitcast_convert_type(lo, jnp.uint32)
    uh = lax.bitcast_convert_type(hi, jnp.uint32)
    rl = (ul + jnp.uint32(0x7FFF) + ((ul >> 16) & 1)) >> 16
    rh = (uh + jnp.uint32(0x7FFF) + ((uh >> 16) & 1)) & jnp.uint32(0xFFFF0000)
    return lax.bitcast_convert_type(rl | rh, jnp.int32)


def _prep_side(q0, q1, m):
    q0 = jnp.clip(q0, LO, HI)
    q1 = jnp.clip(q1, LO, HI)
    f0 = q0.astype(jnp.int32)
    fr0 = q0 - f0.astype(jnp.float32)
    wf0 = 1.0 - fr0
    nz0 = fr0 != 0.0
    wc0 = jnp.where(nz0, fr0, jnp.float32(1.0))
    f1 = q1.astype(jnp.int32)
    fr1 = q1 - f1.astype(jnp.float32)
    wf1 = 1.0 - fr1
    nz1 = fr1 != 0.0
    wc1 = jnp.where(nz1, fr1, jnp.float32(1.0))
    K = ((f1 + IMG * f0)
         | jnp.where(nz1, D1BIT, jnp.int32(0))
         | jnp.where(nz0, D0BIT, jnp.int32(0)))
    A = _pack_bf16_pair(wf0 * m, wc0 * m)
    B = _pack_bf16_pair(wf1, wc1)
    return K, A, B


_BBATCH = 8
_BPIX = PIX // 7


def _prep_body(rowr, colr, c0r, c1r, m1r, m2r,
               kar, aar, bar, kbr, abr, bbr):
    row = rowr[...][None, :]
    col = colr[...][None, :]
    c0 = c0r[...]
    c1 = c1r[...]
    ka, aa, ba = _prep_side(row + c0, col + c1, m1r[...])
    kb, ab, bb = _prep_side(row - c0, col - c1, m2r[...])
    kar[...] = ka
    aar[...] = aa
    bar[...] = ba
    kbr[...] = kb
    abr[...] = ab
    bbr[...] = bb


def _make_prep(nb):
    vspec = pl.BlockSpec((_BPIX,), lambda n, p: (p,))
    bspec = pl.BlockSpec((_BBATCH, _BPIX), lambda n, p: (n, p))
    ospec = jax.ShapeDtypeStruct((nb, PIX), jnp.int32)
    return pl.pallas_call(
        _prep_body,
        grid=(nb // _BBATCH, PIX // _BPIX),
        in_specs=[vspec, vspec, bspec, bspec, bspec, bspec],
        out_specs=[bspec] * 6,
        out_shape=[ospec] * 6,
    )


def _sample(plane, K, A, B):
    iff = K & 0xFFFF
    d1 = (K >> 16) & 1
    d0s = lax.shift_right_logical(K, 24)
    icf = iff + d0s
    ifc = iff + d1
    icc = icf + d1
    wf0m = plsc.bitcast(A << 16, jnp.float32)
    wc0m = plsc.bitcast(A, jnp.float32)
    wf1 = plsc.bitcast(B << 16, jnp.float32)
    wc1 = plsc.bitcast(B, jnp.float32)
    g_ff = plsc.load_gather(plane, [iff])
    g_cf = plsc.load_gather(plane, [icf])
    g_fc = plsc.load_gather(plane, [ifc])
    g_cc = plsc.load_gather(plane, [icc])
    return (wf1 * (wf0m * g_ff + wc0m * g_cf)
            + wc1 * (wf0m * g_fc + wc0m * g_cc))


def _body(tpw, im1, im2, Ka, Aa, Ba, Kb, Ab, Bb, out,
          plane1, plane2,
          ka0, aa0, ba0, kb0, ab0, bb0,
          ka1, aa1, ba1, kb1, ab1, bb1,
          outb0, outb1,
          sem_pl, sem_in0, sem_in1, sem_out0, sem_out1):
    wid = lax.axis_index("s") * 2 + lax.axis_index("c")

    srcs = (Ka, Aa, Ba, Kb, Ab, Bb)
    inbufs = ((ka0, aa0, ba0, kb0, ab0, bb0), (ka1, aa1, ba1, kb1, ab1, bb1))
    outbufs = (outb0, outb1)
    sem_in = (sem_in0, sem_in1)
    sem_out = (sem_out0, sem_out1)

    def do_task(t, carry):
        task = wid * tpw + t
        n = task // NCH

        def fire_in(k, b):
            off = k * P
            for src, ref in zip(srcs, inbufs[b]):
                pltpu.async_copy(src.at[n, pl.ds(off, P)], ref, sem_in[b])

        def drain_in(b):
            for ref in inbufs[b]:
                pltpu.make_async_copy(Ka.at[n, pl.ds(0, P)], ref,
                                      sem_in[b]).wait()

        def wait_out(b):
            pltpu.make_async_copy(outbufs[b],
                                  out.at[task, pl.ds(0, P)],
                                  sem_out[b]).wait()

        cp1 = pltpu.async_copy(im1.at[task], plane1, sem_pl)
        cp2 = pltpu.async_copy(im2.at[task], plane2, sem_pl)
        fire_in(0, 0)
        cp1.wait()
        cp2.wait()

        def do_pair(kk, carry):
            for b in range(2):
                k = kk * 2 + b
                if b == 0:
                    fire_in(k + 1, 1 - b)
                else:
                    @pl.when(kk < (NCHUNK // 2 - 1))
                    def _():
                        fire_in(k + 1, 1 - b)
                drain_in(b)

                @pl.when(kk >= 1)
                def _():
                    wait_out(b)

                kab, aab, bab, kbb, abb, bbb = inbufs[b]
                outb = outbufs[b]
                off = k * P

                @plsc.parallel_loop(0, P, 16, unroll=2)
                def _(j):
                    a = _sample(plane1, kab[pl.ds(j, 16)],
                                aab[pl.ds(j, 16)], bab[pl.ds(j, 16)])
                    b_ = _sample(plane2, kbb[pl.ds(j, 16)],
                                 abb[pl.ds(j, 16)], bbb[pl.ds(j, 16)])
                    outb[pl.ds(j, 16)] = a + b_

                pltpu.async_copy(outb, out.at[task, pl.ds(off, P)],
                                 sem_out[b])
            return carry

        lax.fori_loop(0, NCHUNK // 2, do_pair, 0)
        wait_out(0)
        wait_out(1)
        return carry

    lax.fori_loop(0, tpw, do_task, 0)


def _make_warp(nb):
    mesh = plsc.VectorSubcoreMesh(core_axis_name="c", subcore_axis_name="s")
    return pl.kernel(
        functools.partial(_body, nb * NCH // NWORK),
        out_type=jax.ShapeDtypeStruct((nb * NCH, PIX), jnp.float32),
        mesh=mesh,
        compiler_params=pltpu.CompilerParams(needs_layout_passes=False),
        scratch_types=[
            pltpu.VMEM((PIX,), jnp.float32),
            pltpu.VMEM((PIX,), jnp.float32),
        ] + [pltpu.VMEM((P,), jnp.int32)] * 12 + [
            pltpu.VMEM((P,), jnp.float32),
            pltpu.VMEM((P,), jnp.float32),
            pltpu.SemaphoreType.DMA,
            pltpu.SemaphoreType.DMA,
            pltpu.SemaphoreType.DMA,
            pltpu.SemaphoreType.DMA,
            pltpu.SemaphoreType.DMA,
        ],
    )


NSPLIT = 2


@jax.jit
def kernel(im1, im2, C, M1, M2):
    i = lax.iota(jnp.int32, PIX)
    rowi = i // IMG
    rowf = rowi.astype(jnp.float32)
    colf = (i - rowi * IMG).astype(jnp.float32)
    nb = NB // NSPLIT
    prep = _make_prep(nb)
    warp = _make_warp(nb)
    halves = []
    for g in range(NSPLIT):
        sl = slice(g * nb, (g + 1) * nb)
        c0f = C[sl, 0].reshape(nb, PIX)
        c1f = C[sl, 1].reshape(nb, PIX)
        M1f = M1[sl].reshape(nb, PIX)
        M2f = M2[sl].reshape(nb, PIX)
        Ka, Aa, Ba, Kb, Ab, Bb = prep(rowf, colf, c0f, c1f, M1f, M2f)
        im1f = im1[sl].reshape(nb * NCH, PIX)
        im2f = im2[sl].reshape(nb * NCH, PIX)
        out = warp(im1f, im2f, Ka, Aa, Ba, Kb, Ab, Bb)
        halves.append(out.reshape(nb, NCH, IMG, IMG))
    if NSPLIT == 1:
        return halves[0]
    return jnp.concatenate(halves, axis=0)


# speedup vs baseline: 1.0401x; 1.0401x over previous
import functools

import numpy as np
import jax
import jax.numpy as jnp
from jax import lax
from jax.experimental import pallas as pl
from jax.experimental.pallas import tpu as pltpu
from jax.experimental.pallas import tpu_sc as plsc

IMG = 224
PIX = IMG * IMG
NB = 64
NCH = 3
NWORK = 32
TASKS = NB * NCH
TPW = TASKS // NWORK
ROWS_PER_CHUNK = 8
P = IMG * ROWS_PER_CHUNK
NCHUNK = PIX // P
LO = np.float32(0.001)
HI = np.float32(IMG - 1.001)
D0BIT = np.int32((224 << 24) - (1 << 32))
D1BIT = np.int32(1 << 16)


def _pack_bf16_pair(lo, hi):
    ul = lax.bitcast_convert_type(lo, jnp.uint32)
    uh = lax.bitcast_convert_type(hi, jnp.uint32)
    rl = (ul + jnp.uint32(0x7FFF) + ((ul >> 16) & 1)) >> 16
    rh = (uh + jnp.uint32(0x7FFF) + ((uh >> 16) & 1)) & jnp.uint32(0xFFFF0000)
    return lax.bitcast_convert_type(rl | rh, jnp.int32)


def _prep_side(q0, q1, m):
    q0 = jnp.clip(q0, LO, HI)
    q1 = jnp.clip(q1, LO, HI)
    f0 = q0.astype(jnp.int32)
    fr0 = q0 - f0.astype(jnp.float32)
    wf0 = 1.0 - fr0
    nz0 = fr0 != 0.0
    wc0 = jnp.where(nz0, fr0, jnp.float32(1.0))
    f1 = q1.astype(jnp.int32)
    fr1 = q1 - f1.astype(jnp.float32)
    wf1 = 1.0 - fr1
    nz1 = fr1 != 0.0
    wc1 = jnp.where(nz1, fr1, jnp.float32(1.0))
    K = ((f1 + IMG * f0)
         | jnp.where(nz1, D1BIT, jnp.int32(0))
         | jnp.where(nz0, D0BIT, jnp.int32(0)))
    A = _pack_bf16_pair(wf0 * m, wc0 * m)
    B = _pack_bf16_pair(wf1, wc1)
    return K, A, B


_BBATCH = 8
_BPIX = PIX // 7


def _prep_body(rowr, colr, c0r, c1r, m1r, m2r,
               kar, aar, bar, kbr, abr, bbr):
    row = rowr[...][None, :]
    col = colr[...][None, :]
    c0 = c0r[...]
    c1 = c1r[...]
    ka, aa, ba = _prep_side(row + c0, col + c1, m1r[...])
    kb, ab, bb = _prep_side(row - c0, col - c1, m2r[...])
    kar[...] = ka
    aar[...] = aa
    bar[...] = ba
    kbr[...] = kb
    abr[...] = ab
    bbr[...] = bb


def _make_prep():
    vspec = pl.BlockSpec((_BPIX,), lambda n, p: (p,))
    bspec = pl.BlockSpec((_BBATCH, _BPIX), lambda n, p: (n, p))
    ospec = jax.ShapeDtypeStruct((NB, PIX), jnp.int32)
    return pl.pallas_call(
        _prep_body,
        grid=(NB // _BBATCH, PIX // _BPIX),
        in_specs=[vspec, vspec, bspec, bspec, bspec, bspec],
        out_specs=[bspec] * 6,
        out_shape=[ospec] * 6,
    )


def _sample(plane, K, A, B):
    iff = K & 0xFFFF
    d1 = (K >> 16) & 1
    d0s = lax.shift_right_logical(K, 24)
    icf = iff + d0s
    ifc = iff + d1
    icc = icf + d1
    wf0m = plsc.bitcast(A << 16, jnp.float32)
    wc0m = plsc.bitcast(A, jnp.float32)
    wf1 = plsc.bitcast(B << 16, jnp.float32)
    wc1 = plsc.bitcast(B, jnp.float32)
    g_ff = plsc.load_gather(plane, [iff])
    g_cf = plsc.load_gather(plane, [icf])
    g_fc = plsc.load_gather(plane, [ifc])
    g_cc = plsc.load_gather(plane, [icc])
    return (wf1 * (wf0m * g_ff + wc0m * g_cf)
            + wc1 * (wf0m * g_fc + wc0m * g_cc))


def _body(im1, im2, Ka, Aa, Ba, Kb, Ab, Bb, out,
          plane1, plane2,
          ka0, aa0, ba0, kb0, ab0, bb0,
          ka1, aa1, ba1, kb1, ab1, bb1,
          outb0, outb1,
          sem_pl, sem_in0, sem_in1, sem_out0, sem_out1):
    wid = lax.axis_index("s") * 2 + lax.axis_index("c")

    srcs = (Ka, Aa, Ba, Kb, Ab, Bb)
    inbufs = ((ka0, aa0, ba0, kb0, ab0, bb0), (ka1, aa1, ba1, kb1, ab1, bb1))
    outbufs = (outb0, outb1)
    sem_in = (sem_in0, sem_in1)
    sem_out = (sem_out0, sem_out1)

    def do_task(t, carry):
        task = wid * TPW + t
        n = task // NCH

        def fire_in(k, b):
            off = k * P
            for src, ref in zip(srcs, inbufs[b]):
                pltpu.async_copy(src.at[n, pl.ds(off, P)], ref, sem_in[b])

        def drain_in(b):
            for ref in inbufs[b]:
                pltpu.make_async_copy(Ka.at[n, pl.ds(0, P)], ref,
                                      sem_in[b]).wait()

        def wait_out(b):
            pltpu.make_async_copy(outbufs[b],
                                  out.at[task, pl.ds(0, P)],
                                  sem_out[b]).wait()

        cp1 = pltpu.async_copy(im1.at[task], plane1, sem_pl)
        cp2 = pltpu.async_copy(im2.at[task], plane2, sem_pl)
        fire_in(0, 0)
        cp1.wait()
        cp2.wait()

        def do_pair(kk, carry):
            for b in range(2):
                k = kk * 2 + b
                if b == 0:
                    fire_in(k + 1, 1 - b)
                else:
                    @pl.when(kk < (NCHUNK // 2 - 1))
                    def _():
                        fire_in(k + 1, 1 - b)
                drain_in(b)

                @pl.when(kk >= 1)
                def _():
                    wait_out(b)

                kab, aab, bab, kbb, abb, bbb = inbufs[b]
                outb = outbufs[b]
                off = k * P

                @plsc.parallel_loop(0, P, 16, unroll=3)
                def _(j):
                    a = _sample(plane1, kab[pl.ds(j, 16)],
                                aab[pl.ds(j, 16)], bab[pl.ds(j, 16)])
                    b_ = _sample(plane2, kbb[pl.ds(j, 16)],
                                 abb[pl.ds(j, 16)], bbb[pl.ds(j, 16)])
                    outb[pl.ds(j, 16)] = a + b_

                pltpu.async_copy(outb, out.at[task, pl.ds(off, P)],
                                 sem_out[b])
            return carry

        lax.fori_loop(0, NCHUNK // 2, do_pair, 0)
        wait_out(0)
        wait_out(1)
        return carry

    lax.fori_loop(0, TPW, do_task, 0)


def _make_warp():
    mesh = plsc.VectorSubcoreMesh(core_axis_name="c", subcore_axis_name="s")
    return pl.kernel(
        _body,
        out_type=jax.ShapeDtypeStruct((NB * NCH, PIX), jnp.float32),
        mesh=mesh,
        compiler_params=pltpu.CompilerParams(needs_layout_passes=False),
        scratch_types=[
            pltpu.VMEM((PIX,), jnp.float32),
            pltpu.VMEM((PIX,), jnp.float32),
        ] + [pltpu.VMEM((P,), jnp.int32)] * 12 + [
            pltpu.VMEM((P,), jnp.float32),
            pltpu.VMEM((P,), jnp.float32),
            pltpu.SemaphoreType.DMA,
            pltpu.SemaphoreType.DMA,
            pltpu.SemaphoreType.DMA,
            pltpu.SemaphoreType.DMA,
            pltpu.SemaphoreType.DMA,
        ],
    )


@jax.jit
def kernel(im1, im2, C, M1, M2):
    c0f = C[:, 0].reshape(NB, PIX)
    c1f = C[:, 1].reshape(NB, PIX)
    M1f = M1.reshape(NB, PIX)
    M2f = M2.reshape(NB, PIX)
    i = lax.iota(jnp.int32, PIX)
    rowi = i // IMG
    rowf = rowi.astype(jnp.float32)
    colf = (i - rowi * IMG).astype(jnp.float32)
    Ka, Aa, Ba, Kb, Ab, Bb = _make_prep()(rowf, colf, c0f, c1f, M1f, M2f)
    im1f = im1.reshape(NB * NCH, PIX)
    im2f = im2.reshape(NB * NCH, PIX)
    out = _make_warp()(im1f, im2f, Ka, Aa, Ba, Kb, Ab, Bb)
    return out.reshape(NB, NCH, IMG, IMG)


# speedup vs baseline: 1.0422x; 1.0020x over previous
import functools

import numpy as np
import jax
import jax.numpy as jnp
from jax import lax
from jax.experimental import pallas as pl
from jax.experimental.pallas import tpu as pltpu
from jax.experimental.pallas import tpu_sc as plsc

IMG = 224
PIX = IMG * IMG
NB = 64
NCH = 3
NWORK = 32
TASKS = NB * NCH
TPW = TASKS // NWORK
ROWS_PER_CHUNK = 8
P = IMG * ROWS_PER_CHUNK
NCHUNK = PIX // P
LO = np.float32(0.001)
HI = np.float32(IMG - 1.001)
D0BIT = np.int32((224 << 24) - (1 << 32))
D1BIT = np.int32(1 << 16)


def _pack_bf16_pair(lo, hi):
    ul = lax.bitcast_convert_type(lo, jnp.uint32)
    uh = lax.bitcast_convert_type(hi, jnp.uint32)
    rl = (ul + jnp.uint32(0x7FFF) + ((ul >> 16) & 1)) >> 16
    rh = (uh + jnp.uint32(0x7FFF) + ((uh >> 16) & 1)) & jnp.uint32(0xFFFF0000)
    return lax.bitcast_convert_type(rl | rh, jnp.int32)


def _prep_side(q0, q1, m):
    q0 = jnp.clip(q0, LO, HI)
    q1 = jnp.clip(q1, LO, HI)
    f0 = q0.astype(jnp.int32)
    fr0 = q0 - f0.astype(jnp.float32)
    wf0 = 1.0 - fr0
    nz0 = fr0 != 0.0
    wc0 = jnp.where(nz0, fr0, jnp.float32(1.0))
    f1 = q1.astype(jnp.int32)
    fr1 = q1 - f1.astype(jnp.float32)
    wf1 = 1.0 - fr1
    nz1 = fr1 != 0.0
    wc1 = jnp.where(nz1, fr1, jnp.float32(1.0))
    K = ((f1 + IMG * f0)
         | jnp.where(nz1, D1BIT, jnp.int32(0))
         | jnp.where(nz0, D0BIT, jnp.int32(0)))
    A = _pack_bf16_pair(wf0 * m, wc0 * m)
    B = _pack_bf16_pair(wf1, wc1)
    return K, A, B


_BBATCH = 8
_BPIX = PIX // 7


def _prep_body(rowr, colr, c0r, c1r, m1r, m2r,
               kar, aar, bar, kbr, abr, bbr):
    row = rowr[...][None, :]
    col = colr[...][None, :]
    c0 = c0r[...]
    c1 = c1r[...]
    ka, aa, ba = _prep_side(row + c0, col + c1, m1r[...])
    kb, ab, bb = _prep_side(row - c0, col - c1, m2r[...])
    kar[...] = ka
    aar[...] = aa
    bar[...] = ba
    kbr[...] = kb
    abr[...] = ab
    bbr[...] = bb


def _make_prep():
    vspec = pl.BlockSpec((_BPIX,), lambda n, p: (p,))
    bspec = pl.BlockSpec((_BBATCH, _BPIX), lambda n, p: (n, p))
    ospec = jax.ShapeDtypeStruct((NB, PIX), jnp.int32)
    return pl.pallas_call(
        _prep_body,
        grid=(NB // _BBATCH, PIX // _BPIX),
        in_specs=[vspec, vspec, bspec, bspec, bspec, bspec],
        out_specs=[bspec] * 6,
        out_shape=[ospec] * 6,
    )


def _sample(plane, K, A, B):
    iff = K & 0xFFFF
    d1 = (K >> 16) & 1
    d0s = lax.shift_right_logical(K, 24)
    icf = iff + d0s
    ifc = iff + d1
    icc = icf + d1
    wf0m = plsc.bitcast(A << 16, jnp.float32)
    wc0m = plsc.bitcast(A, jnp.float32)
    wf1 = plsc.bitcast(B << 16, jnp.float32)
    wc1 = plsc.bitcast(B, jnp.float32)
    g_ff = plsc.load_gather(plane, [iff])
    g_cf = plsc.load_gather(plane, [icf])
    g_fc = plsc.load_gather(plane, [ifc])
    g_cc = plsc.load_gather(plane, [icc])
    return (wf1 * (wf0m * g_ff + wc0m * g_cf)
            + wc1 * (wf0m * g_fc + wc0m * g_cc))


def _body(im1, im2, Ka, Aa, Ba, Kb, Ab, Bb, out,
          plane1, plane2,
          inb0, inb1,
          outb0, outb1,
          sem_pl, sem_in0, sem_in1, sem_out0, sem_out1):
    wid = lax.axis_index("s") * 2 + lax.axis_index("c")

    srcs = (Ka, Aa, Ba, Kb, Ab, Bb)
    inbufs = (inb0, inb1)
    outbufs = (outb0, outb1)
    sem_in = (sem_in0, sem_in1)
    sem_out = (sem_out0, sem_out1)

    def do_task(t, carry):
        task = wid * TPW + t
        n = task // NCH

        def fire_in(k, b):
            off = k * P
            for i, src in enumerate(srcs):
                pltpu.async_copy(src.at[n, pl.ds(off, P)],
                                 inbufs[b].at[pl.ds(i * P, P)], sem_in[b])

        def drain_in(b):
            pltpu.make_async_copy(Ka.at[n, pl.ds(0, 6 * P)],
                                  inbufs[b], sem_in[b]).wait()

        def wait_out(b):
            pltpu.make_async_copy(outbufs[b],
                                  out.at[task, pl.ds(0, P)],
                                  sem_out[b]).wait()

        cp1 = pltpu.async_copy(im1.at[task], plane1, sem_pl)
        cp2 = pltpu.async_copy(im2.at[task], plane2, sem_pl)
        fire_in(0, 0)
        cp1.wait()
        cp2.wait()

        def do_pair(kk, carry):
            for b in range(2):
                k = kk * 2 + b
                if b == 0:
                    fire_in(k + 1, 1 - b)
                else:
                    @pl.when(kk < (NCHUNK // 2 - 1))
                    def _():
                        fire_in(k + 1, 1 - b)
                drain_in(b)

                @pl.when(kk >= 1)
                def _():
                    wait_out(b)

                inb = inbufs[b]
                outb = outbufs[b]
                off = k * P

                @plsc.parallel_loop(0, P, 16, unroll=3)
                def _(j):
                    a = _sample(plane1, inb[pl.ds(j, 16)],
                                inb[pl.ds(P + j, 16)],
                                inb[pl.ds(2 * P + j, 16)])
                    b_ = _sample(plane2, inb[pl.ds(3 * P + j, 16)],
                                 inb[pl.ds(4 * P + j, 16)],
                                 inb[pl.ds(5 * P + j, 16)])
                    outb[pl.ds(j, 16)] = a + b_

                pltpu.async_copy(outb, out.at[task, pl.ds(off, P)],
                                 sem_out[b])
            return carry

        lax.fori_loop(0, NCHUNK // 2, do_pair, 0)
        wait_out(0)
        wait_out(1)
        return carry

    lax.fori_loop(0, TPW, do_task, 0)


def _make_warp():
    mesh = plsc.VectorSubcoreMesh(core_axis_name="c", subcore_axis_name="s")
    return pl.kernel(
        _body,
        out_type=jax.ShapeDtypeStruct((NB * NCH, PIX), jnp.float32),
        mesh=mesh,
        compiler_params=pltpu.CompilerParams(needs_layout_passes=False),
        scratch_types=[
            pltpu.VMEM((PIX,), jnp.float32),
            pltpu.VMEM((PIX,), jnp.float32),
        ] + [pltpu.VMEM((6 * P,), jnp.int32)] * 2 + [
            pltpu.VMEM((P,), jnp.float32),
            pltpu.VMEM((P,), jnp.float32),
            pltpu.SemaphoreType.DMA,
            pltpu.SemaphoreType.DMA,
            pltpu.SemaphoreType.DMA,
            pltpu.SemaphoreType.DMA,
            pltpu.SemaphoreType.DMA,
        ],
    )


@jax.jit
def kernel(im1, im2, C, M1, M2):
    c0f = C[:, 0].reshape(NB, PIX)
    c1f = C[:, 1].reshape(NB, PIX)
    M1f = M1.reshape(NB, PIX)
    M2f = M2.reshape(NB, PIX)
    i = lax.iota(jnp.int32, PIX)
    rowi = i // IMG
    rowf = rowi.astype(jnp.float32)
    colf = (i - rowi * IMG).astype(jnp.float32)
    Ka, Aa, Ba, Kb, Ab, Bb = _make_prep()(rowf, colf, c0f, c1f, M1f, M2f)
    im1f = im1.reshape(NB * NCH, PIX)
    im2f = im2.reshape(NB * NCH, PIX)
    out = _make_warp()(im1f, im2f, Ka, Aa, Ba, Kb, Ab, Bb)
    return out.reshape(NB, NCH, IMG, IMG)
